# baseline (speedup 1.0000x reference)
import functools

import jax
import jax.numpy as jnp
from jax import lax
from jax.experimental import pallas as pl
from jax.experimental.pallas import tpu as pltpu
from jax.experimental.pallas import tpu_sc as plsc

B, T, C = 64, 2048, 64
MAXR, MAXC = 11, 11
CPAD = 16
N = T * C
GS = 8
NW = 32
NTASK = MAXC * (B // GS)
LANES = 16


def _gather_body(src_ref, x_ref, comp_ref, st_ref):
    xb = x_ref[...]
    s1 = jnp.sum(xb, axis=(1, 2))
    s2 = jnp.sum(xb * xb, axis=(1, 2))
    st_ref[0] = jnp.broadcast_to(s1[:, None], (GS, 128))
    st_ref[1] = jnp.broadcast_to(s2[:, None], (GS, 128))
    iota_c = jax.lax.broadcasted_iota(jnp.int32, (CPAD, C), 1)
    w = (iota_c == src_ref[0][:, None]).astype(jnp.bfloat16)
    hi = xb.astype(jnp.bfloat16)
    lo = (xb - hi.astype(jnp.float32)).astype(jnp.bfloat16)
    dn = (((1,), (1,)), ((), ()))
    y = jax.lax.dot_general(w, hi, dn, preferred_element_type=jnp.float32)
    y = y + jax.lax.dot_general(w, lo, dn, preferred_element_type=jnp.float32)
    comp_ref[...] = y


def _sc_body(comp_hbm, st_hbm, yo_hbm, data_v, s1_v, s2_v):
    cid = lax.axis_index("c")
    sid = lax.axis_index("s")
    wid = sid * 2 + cid

    for i in range(3):
        tid = wid + NW * i

        @pl.when(tid < NTASK)
        def _task():
            cg = tid % MAXC
            g = tid // MAXC
            pltpu.sync_copy(comp_hbm.at[cg, pl.ds(GS * g, GS), :], data_v)
            pltpu.sync_copy(st_hbm.at[0, pl.ds(GS * g, GS), :], s1_v)
            pltpu.sync_copy(st_hbm.at[1, pl.ds(GS * g, GS), :], s2_v)
            for s in range(GS):
                s1 = s1_v[s, pl.ds(0, LANES)]
                s2 = s2_v[s, pl.ds(0, LANES)]
                mean = s1 * (1.0 / N)
                var = (s2 - s1 * mean) * (1.0 / (N - 1))
                y = plsc.bitcast(
                    0x5F3759DF - (plsc.bitcast(var, jnp.int32) >> 1),
                    jnp.float32)
                for _ in range(3):
                    y = y * (1.5 - 0.5 * var * y * y)
                stdv = var * y
                av = 1.0 / (stdv + 1e-6)
                bv = -mean * av

                @plsc.parallel_loop(0, T // LANES, step=1, unroll=8)
                def _chunk(k):
                    v = data_v[s, pl.ds(k * LANES, LANES)]
                    data_v[s, pl.ds(k * LANES, LANES)] = v * av + bv

            for r in range(MAXR):
                pltpu.sync_copy(data_v,
                                yo_hbm.at[r, cg, pl.ds(GS * g, GS), :])


def kernel(x, rearrange_idx, original_idx):
    src = jnp.zeros((MAXC,), jnp.int32).at[rearrange_idx].set(original_idx)
    src16 = jnp.pad(src, (0, CPAD - MAXC)).reshape(1, CPAD)
    xt = jnp.transpose(x, (0, 2, 1))

    comp, st = pl.pallas_call(
        _gather_body,
        grid=(B // GS,),
        in_specs=[
            pl.BlockSpec((1, CPAD), lambda g: (0, 0)),
            pl.BlockSpec((GS, C, T), lambda g: (g, 0, 0)),
        ],
        out_specs=[
            pl.BlockSpec((CPAD, GS, T), lambda g: (0, g, 0)),
            pl.BlockSpec((2, GS, 128), lambda g: (0, g, 0)),
        ],
        out_shape=[
            jax.ShapeDtypeStruct((CPAD, B, T), jnp.float32),
            jax.ShapeDtypeStruct((2, B, 128), jnp.float32),
        ],
    )(src16, xt)

    mesh = plsc.VectorSubcoreMesh(core_axis_name="c", subcore_axis_name="s")
    f = functools.partial(
        pl.kernel,
        mesh=mesh,
        compiler_params=pltpu.CompilerParams(
            needs_layout_passes=False, use_tc_tiling_on_sc=True),
        out_type=jax.ShapeDtypeStruct((MAXR, MAXC, B, T), jnp.float32),
        scratch_types=[
            pltpu.VMEM((GS, T), jnp.float32),
            pltpu.VMEM((GS, 128), jnp.float32),
            pltpu.VMEM((GS, 128), jnp.float32),
        ],
    )(_sc_body)
    yo = f(comp, st)

    return jnp.transpose(yo, (2, 3, 0, 1))


# speedup vs baseline: 1.1045x; 1.1045x over previous
import functools

import jax
import jax.numpy as jnp
from jax import lax
from jax.experimental import pallas as pl
from jax.experimental.pallas import tpu as pltpu
from jax.experimental.pallas import tpu_sc as plsc

B, T, C = 64, 2048, 64
MAXR, MAXC = 11, 11
CPAD = 16
N = T * C
GS = 8
NW = 32
NTASK = MAXC * (B // GS)


def _gather_body(src_ref, x_ref, comp_ref):
    xb = x_ref[...]
    s1 = jnp.sum(xb, axis=(1, 2))
    s2 = jnp.sum(xb * xb, axis=(1, 2))
    mean = s1 * (1.0 / N)
    var = (s2 - s1 * mean) * (1.0 / (N - 1))
    rstd = 1.0 / (jnp.sqrt(var) + 1e-6)
    iota_c = jax.lax.broadcasted_iota(jnp.int32, (CPAD, C), 1)
    w = (iota_c == src_ref[0][:, None]).astype(jnp.bfloat16)
    hi = xb.astype(jnp.bfloat16)
    lo = (xb - hi.astype(jnp.float32)).astype(jnp.bfloat16)
    dn = (((1,), (1,)), ((), ()))
    y = jax.lax.dot_general(w, hi, dn, preferred_element_type=jnp.float32)
    y = y + jax.lax.dot_general(w, lo, dn, preferred_element_type=jnp.float32)
    comp_ref[...] = (y - mean[None, :, None]) * rstd[None, :, None]


def _sc_body(comp_hbm, yo_hbm, data_v, sem):
    cid = lax.axis_index("c")
    sid = lax.axis_index("s")
    wid = sid * 2 + cid

    for i in range(3):
        tid = wid + NW * i

        @pl.when(tid < NTASK)
        def _task():
            cg = tid % MAXC
            g = tid // MAXC
            pltpu.sync_copy(comp_hbm.at[cg, pl.ds(GS * g, GS), :], data_v)
            copies = [
                pltpu.async_copy(
                    data_v, yo_hbm.at[r, cg, pl.ds(GS * g, GS), :], sem)
                for r in range(MAXR)
            ]
            for cp in copies:
                cp.wait()


def kernel(x, rearrange_idx, original_idx):
    src = jnp.zeros((MAXC,), jnp.int32).at[rearrange_idx].set(original_idx)
    src16 = jnp.pad(src, (0, CPAD - MAXC)).reshape(1, CPAD)
    xt = jnp.transpose(x, (0, 2, 1))

    comp = pl.pallas_call(
        _gather_body,
        grid=(B // GS,),
        in_specs=[
            pl.BlockSpec((1, CPAD), lambda g: (0, 0)),
            pl.BlockSpec((GS, C, T), lambda g: (g, 0, 0)),
        ],
        out_specs=pl.BlockSpec((CPAD, GS, T), lambda g: (0, g, 0)),
        out_shape=jax.ShapeDtypeStruct((CPAD, B, T), jnp.float32),
    )(src16, xt)

    mesh = plsc.VectorSubcoreMesh(core_axis_name="c", subcore_axis_name="s")
    f = functools.partial(
        pl.kernel,
        mesh=mesh,
        compiler_params=pltpu.CompilerParams(
            needs_layout_passes=False, use_tc_tiling_on_sc=True),
        out_type=jax.ShapeDtypeStruct((MAXR, MAXC, B, T), jnp.float32),
        scratch_types=[
            pltpu.VMEM((GS, T), jnp.float32),
            pltpu.SemaphoreType.DMA,
        ],
    )(_sc_body)
    yo = f(comp)

    return jnp.transpose(yo, (2, 3, 0, 1))


# speedup vs baseline: 1.1163x; 1.0107x over previous
import functools

import jax
import jax.numpy as jnp
from jax import lax
from jax.experimental import pallas as pl
from jax.experimental.pallas import tpu as pltpu
from jax.experimental.pallas import tpu_sc as plsc

B, T, C = 64, 2048, 64
MAXR, MAXC = 11, 11
CPAD = 16
N = T * C
GS = 8
NW = 32
NTASK = MAXC * (B // GS)


def _gather_body(src_ref, x_ref, comp_ref):
    xb = x_ref[...]
    s1 = jnp.sum(xb, axis=(1, 2))
    s2 = jnp.sum(xb * xb, axis=(1, 2))
    mean = s1 * (1.0 / N)
    var = (s2 - s1 * mean) * (1.0 / (N - 1))
    rstd = 1.0 / (jnp.sqrt(var) + 1e-6)
    iota_c = jax.lax.broadcasted_iota(jnp.int32, (CPAD, C), 1)
    w = (iota_c == src_ref[0][:, None]).astype(jnp.bfloat16)
    hi = xb.astype(jnp.bfloat16)
    lo = (xb - hi.astype(jnp.float32)).astype(jnp.bfloat16)
    dn = (((1,), (1,)), ((), ()))
    y = jax.lax.dot_general(w, hi, dn, preferred_element_type=jnp.float32)
    y = y + jax.lax.dot_general(w, lo, dn, preferred_element_type=jnp.float32)
    comp_ref[...] = (y - mean[None, :, None]) * rstd[None, :, None]


def _sc_body(comp_hbm, yo_hbm, data0_v, data1_v, sem0, sem1):
    cid = lax.axis_index("c")
    sid = lax.axis_index("s")
    wid = sid * 2 + cid

    bufs = (data0_v, data1_v)
    sems = (sem0, sem1)
    pending = [None, None]
    for i in range(3):
        tid = (wid + NW * i) % NTASK
        cg = tid % MAXC
        g = tid // MAXC
        p = i % 2
        if pending[p] is not None:
            for cp in pending[p]:
                cp.wait()
        pltpu.sync_copy(comp_hbm.at[cg, pl.ds(GS * g, GS), :], bufs[p])
        pending[p] = [
            pltpu.async_copy(
                bufs[p], yo_hbm.at[r, cg, pl.ds(GS * g, GS), :], sems[p])
            for r in range(MAXR)
        ]
    for p in range(2):
        if pending[p] is not None:
            for cp in pending[p]:
                cp.wait()


def kernel(x, rearrange_idx, original_idx):
    src = jnp.zeros((MAXC,), jnp.int32).at[rearrange_idx].set(original_idx)
    src16 = jnp.pad(src, (0, CPAD - MAXC)).reshape(1, CPAD)
    xt = jnp.transpose(x, (0, 2, 1))

    comp = pl.pallas_call(
        _gather_body,
        grid=(B // GS,),
        in_specs=[
            pl.BlockSpec((1, CPAD), lambda g: (0, 0)),
            pl.BlockSpec((GS, C, T), lambda g: (g, 0, 0)),
        ],
        out_specs=pl.BlockSpec((CPAD, GS, T), lambda g: (0, g, 0)),
        out_shape=jax.ShapeDtypeStruct((CPAD, B, T), jnp.float32),
    )(src16, xt)

    mesh = plsc.VectorSubcoreMesh(core_axis_name="c", subcore_axis_name="s")
    f = functools.partial(
        pl.kernel,
        mesh=mesh,
        compiler_params=pltpu.CompilerParams(
            needs_layout_passes=False, use_tc_tiling_on_sc=True),
        out_type=jax.ShapeDtypeStruct((MAXR, MAXC, B, T), jnp.float32),
        scratch_types=[
            pltpu.VMEM((GS, T), jnp.float32),
            pltpu.VMEM((GS, T), jnp.float32),
            pltpu.SemaphoreType.DMA,
            pltpu.SemaphoreType.DMA,
        ],
    )(_sc_body)
    yo = f(comp)

    return jnp.transpose(yo, (2, 3, 0, 1))
